# baseline (speedup 1.0000x reference)
import jax
import jax.numpy as jnp
from jax import lax
from jax.experimental import pallas as pl
from jax.experimental.pallas import tpu as pltpu
from jax.experimental.pallas import tpu_sc as plsc

NC = 2
NS = 16
NW = NC * NS
LANES = 16
D = 256
W = 8
WP = 16
CH = 128
CHB = 8
MAIN = 13

_SC_PARAMS = pltpu.CompilerParams(use_tc_tiling_on_sc=False)


def _sc_scatter_ids(nch, tsize):
    mesh = plsc.VectorSubcoreMesh(core_axis_name="c", subcore_axis_name="s")

    def body(keys, ids, t_out, keys_v, ids_v, sem):
        wid = lax.axis_index("s") * NC + lax.axis_index("c")
        pltpu.sync_copy(keys.at[wid], keys_v)
        pltpu.sync_copy(ids.at[wid], ids_v)

        def fire(c, _):
            pltpu.async_copy(ids_v.at[c], t_out.at[keys_v.at[c]], sem)
            return 0

        def drain(c, _):
            pltpu.make_async_copy(ids_v.at[c], t_out.at[keys_v.at[c]], sem).wait()
            return 0

        lax.fori_loop(0, nch, fire, 0)
        lax.fori_loop(0, nch, drain, 0)

    return pl.kernel(
        body,
        out_type=jax.ShapeDtypeStruct((tsize,), jnp.int32),
        mesh=mesh,
        scratch_types=[
            pltpu.VMEM((nch, CH), jnp.int32),
            pltpu.VMEM((nch, CH), jnp.int32),
            pltpu.SemaphoreType.DMA,
        ],
        compiler_params=_SC_PARAMS,
    )


def _sc_xor_count(nch, n_edges):
    mesh = plsc.VectorSubcoreMesh(core_axis_name="c", subcore_axis_name="s")

    def body(tab, t_in, keys, ids, sa, sb, out, uout,
             keys_v, ids_v, sa_v, sb_v, tval_v, ia_v, ib_v,
             ba, bb, cnt_v, semt, sem0, sem1):
        wid = lax.axis_index("s") * NC + lax.axis_index("c")
        pltpu.sync_copy(keys.at[wid], keys_v)
        pltpu.sync_copy(ids.at[wid], ids_v)
        pltpu.sync_copy(sa.at[wid], sa_v)
        pltpu.sync_copy(sb.at[wid], sb_v)

        def tfire(c, _):
            pltpu.async_copy(t_in.at[keys_v.at[c]], tval_v.at[c], semt)
            return 0

        lax.fori_loop(0, nch, tfire, 0)

        zero = jnp.zeros((LANES,), jnp.int32)
        ucnt0 = zero

        def tdrain(c, ucnt):
            pltpu.make_async_copy(t_in.at[keys_v.at[c]], tval_v.at[c], semt).wait()
            for j in range(CH // LANES):
                s = pl.ds(j * LANES, LANES)
                win = tval_v[c, s] == ids_v[c, s]
                ia_v[c, s] = jnp.where(win, sa_v[c, s], 0)
                ib_v[c, s] = jnp.where(win, sb_v[c, s], 0)
                real = jnp.logical_and(win, ids_v[c, s] < n_edges)
                ucnt = ucnt + jnp.where(real, 1, 0)
            return ucnt

        ucnt = lax.fori_loop(0, nch, tdrain, ucnt0)
        cnt_v[pl.ds(0, LANES)] = ucnt
        pltpu.sync_copy(cnt_v.at[pl.ds(0, LANES)], uout.at[wid])

        sems = [sem0, sem1]

        def fire(c, par):
            pltpu.async_copy(tab.at[ia_v.at[c]], ba.at[par], sems[par])
            pltpu.async_copy(tab.at[ib_v.at[c]], bb.at[par], sems[par])

        def drain(c, par):
            pltpu.make_async_copy(tab.at[ia_v.at[c]], ba.at[par], sems[par]).wait()
            pltpu.make_async_copy(tab.at[ib_v.at[c]], bb.at[par], sems[par]).wait()

        fire(0, 0)
        main0 = (zero,) * MAIN

        def pair_body(i, main):
            for par in range(2):
                c = 2 * i + par
                drain(c, par)

                @pl.when(c + 1 < nch)
                def _():
                    fire(c + 1, 1 - par)

                main = list(main)
                ch = [zero] * CHB
                for j in range(CH):
                    carry = lax.bitwise_xor(ba[par, j, :], bb[par, j, :])
                    for k in range((j + 1).bit_length()):
                        t = lax.bitwise_and(ch[k], carry)
                        ch[k] = lax.bitwise_xor(ch[k], carry)
                        carry = t
                for k in range(CHB):
                    carry = ch[k]
                    for l in range(k, MAIN):
                        t = lax.bitwise_and(main[l], carry)
                        main[l] = lax.bitwise_xor(main[l], carry)
                        carry = t
                main = tuple(main)
            return main

        main = lax.fori_loop(0, nch // 2, pair_body, main0)

        for b in range(32):
            cnt = zero
            for k in range(MAIN):
                bit = lax.bitwise_and(lax.shift_right_logical(main[k], b), 1)
                cnt = cnt + lax.shift_left(bit, k)
            cnt_v[pl.ds(b * LANES, LANES)] = cnt
        pltpu.sync_copy(cnt_v, out.at[wid])

    return pl.kernel(
        body,
        out_type=(
            jax.ShapeDtypeStruct((NW, 32 * LANES), jnp.int32),
            jax.ShapeDtypeStruct((NW, LANES), jnp.int32),
        ),
        mesh=mesh,
        scratch_types=[
            pltpu.VMEM((nch, CH), jnp.int32),
            pltpu.VMEM((nch, CH), jnp.int32),
            pltpu.VMEM((nch, CH), jnp.int32),
            pltpu.VMEM((nch, CH), jnp.int32),
            pltpu.VMEM((nch, CH), jnp.int32),
            pltpu.VMEM((nch, CH), jnp.int32),
            pltpu.VMEM((nch, CH), jnp.int32),
            pltpu.VMEM((2, CH, WP), jnp.int32),
            pltpu.VMEM((2, CH, WP), jnp.int32),
            pltpu.VMEM((32 * LANES,), jnp.int32),
            pltpu.SemaphoreType.DMA,
            pltpu.SemaphoreType.DMA,
            pltpu.SemaphoreType.DMA,
        ],
        compiler_params=_SC_PARAMS,
    )


def _tc_reduce_am(part_ref, u_ref, am_ref, out_ref):
    cnt = jnp.sum(part_ref[...], axis=0, keepdims=True)
    u = jnp.sum(u_ref[...]).astype(jnp.float32)
    enc = u - 2.0 * cnt.astype(jnp.float32)
    out_ref[...] = lax.dot_general(
        enc, am_ref[...], (((1,), (1,)), ((), ())),
        preferred_element_type=jnp.float32,
    )


def kernel(x, edge_index, pr, ids_weight, am_weight):
    n = x.shape[0]
    e = edge_index.shape[1]

    bits = (ids_weight[:n] < 0).reshape(n, W, 32).astype(jnp.int32)
    words = jnp.sum(
        jnp.left_shift(bits, jnp.arange(32, dtype=jnp.int32)), axis=-1
    )
    words = jnp.concatenate(
        [words, jnp.zeros((n, WP - W), jnp.int32)], axis=1
    )
    pr_argsort = jnp.argsort(pr)
    ptab = jnp.zeros((n, WP), jnp.int32).at[pr_argsort].set(words)

    a = jnp.minimum(edge_index[0], edge_index[1])
    b = jnp.maximum(edge_index[0], edge_index[1])
    keys = a * n + b
    padkey = n * n

    nch = -(-e // (NW * CH))
    if nch % 2:
        nch += 1
    e_pad = NW * nch * CH
    pad = e_pad - e
    keys = jnp.concatenate([keys, jnp.full((pad,), padkey, jnp.int32)])
    sa = jnp.concatenate([a, jnp.zeros((pad,), jnp.int32)])
    sb = jnp.concatenate([b, jnp.zeros((pad,), jnp.int32)])
    ids = jnp.arange(e_pad, dtype=jnp.int32)
    shape3 = (NW, nch, CH)
    keys = keys.reshape(shape3)
    sa = sa.reshape(shape3)
    sb = sb.reshape(shape3)
    ids = ids.reshape(shape3)

    tsize = n * n + 8
    t = _sc_scatter_ids(nch, tsize)(keys, ids)
    partials, upart = _sc_xor_count(nch, e)(ptab, t, keys, ids, sa, sb)
    partials = (
        partials.reshape(NW, 32, LANES)
        .transpose(0, 2, 1)[:, :W, :]
        .reshape(NW, D)
    )

    scores = pl.pallas_call(
        _tc_reduce_am,
        out_shape=jax.ShapeDtypeStruct((1, am_weight.shape[0]), jnp.float32),
    )(partials, upart, am_weight)
    return scores


# speedup vs baseline: 2.3903x; 2.3903x over previous
import jax
import jax.numpy as jnp
from jax import lax
from jax.experimental import pallas as pl
from jax.experimental.pallas import tpu as pltpu
from jax.experimental.pallas import tpu_sc as plsc

NC = 2
NS = 16
NW = NC * NS
LANES = 16
D = 256
W = 8
WP = 16
CH = 128
CHB = 8
MAIN = 13


def _sc_xor_count(nch):
    mesh = plsc.VectorSubcoreMesh(core_axis_name="c", subcore_axis_name="s")

    def body(tab, ia, ib, out, ia_v, ib_v, ba, bb, cnt_v, sem0, sem1):
        wid = lax.axis_index("s") * NC + lax.axis_index("c")
        pltpu.sync_copy(ia.at[wid], ia_v)
        pltpu.sync_copy(ib.at[wid], ib_v)
        sems = [sem0, sem1]

        def fire(c, par):
            pltpu.async_copy(tab.at[ia_v.at[c]], ba.at[par], sems[par])
            pltpu.async_copy(tab.at[ib_v.at[c]], bb.at[par], sems[par])

        def drain(c, par):
            pltpu.make_async_copy(tab.at[ia_v.at[c]], ba.at[par], sems[par]).wait()
            pltpu.make_async_copy(tab.at[ib_v.at[c]], bb.at[par], sems[par]).wait()

        fire(0, 0)

        zero = jnp.zeros((LANES,), jnp.int32)
        main0 = (zero,) * MAIN

        def pair_body(i, main):
            for par in range(2):
                c = 2 * i + par
                drain(c, par)

                @pl.when(c + 1 < nch)
                def _():
                    fire(c + 1, 1 - par)

                main = list(main)
                ch = [zero] * CHB
                for j in range(CH):
                    carry = lax.bitwise_xor(ba[par, j, :], bb[par, j, :])
                    for k in range((j + 1).bit_length()):
                        t = lax.bitwise_and(ch[k], carry)
                        ch[k] = lax.bitwise_xor(ch[k], carry)
                        carry = t
                for k in range(CHB):
                    carry = ch[k]
                    for l in range(k, MAIN):
                        t = lax.bitwise_and(main[l], carry)
                        main[l] = lax.bitwise_xor(main[l], carry)
                        carry = t
                main = tuple(main)
            return main

        main = lax.fori_loop(0, nch // 2, pair_body, main0)

        for b in range(32):
            cnt = zero
            for k in range(MAIN):
                bit = lax.bitwise_and(lax.shift_right_logical(main[k], b), 1)
                cnt = cnt + lax.shift_left(bit, k)
            cnt_v[pl.ds(b * LANES, LANES)] = cnt
        pltpu.sync_copy(cnt_v, out.at[wid])

    return pl.kernel(
        body,
        out_type=jax.ShapeDtypeStruct((NW, 32 * LANES), jnp.int32),
        mesh=mesh,
        scratch_types=[
            pltpu.VMEM((nch, CH), jnp.int32),
            pltpu.VMEM((nch, CH), jnp.int32),
            pltpu.VMEM((2, CH, WP), jnp.int32),
            pltpu.VMEM((2, CH, WP), jnp.int32),
            pltpu.VMEM((32 * LANES,), jnp.int32),
            pltpu.SemaphoreType.DMA,
            pltpu.SemaphoreType.DMA,
        ],
        compiler_params=pltpu.CompilerParams(use_tc_tiling_on_sc=False),
    )


def _tc_reduce_am(part_ref, u_ref, am_ref, out_ref):
    cnt = jnp.sum(part_ref[...], axis=0, keepdims=True)
    enc = u_ref[...] - 2.0 * cnt.astype(jnp.float32)
    out_ref[...] = lax.dot_general(
        enc, am_ref[...], (((1,), (1,)), ((), ())),
        preferred_element_type=jnp.float32,
    )


def kernel(x, edge_index, pr, ids_weight, am_weight):
    n = x.shape[0]
    e = edge_index.shape[1]

    bits = (ids_weight[:n] < 0).reshape(n, W, 32).astype(jnp.int32)
    words = jnp.sum(
        jnp.left_shift(bits, jnp.arange(32, dtype=jnp.int32)), axis=-1
    )
    words = jnp.concatenate(
        [words, jnp.zeros((n, WP - W), jnp.int32)], axis=1
    )
    pr_argsort = jnp.argsort(pr)
    ptab = jnp.zeros((n, WP), jnp.int32).at[pr_argsort].set(words)

    a = jnp.minimum(edge_index[0], edge_index[1])
    b = jnp.maximum(edge_index[0], edge_index[1])
    keys = a * n + b
    ks = lax.sort(keys)
    sa = ks // jnp.int32(n)
    sb = ks - sa * jnp.int32(n)
    first = jnp.concatenate(
        [jnp.ones((1,), dtype=bool), ks[1:] != ks[:-1]]
    )
    ia = jnp.where(first, sa, 0)
    ib = jnp.where(first, sb, 0)
    u = jnp.sum(first, dtype=jnp.int32).astype(jnp.float32).reshape(1, 1)

    nch = -(-e // (NW * CH))
    if nch % 2:
        nch += 1
    e_pad = NW * nch * CH
    ia = jnp.concatenate([ia, jnp.zeros((e_pad - e,), jnp.int32)])
    ib = jnp.concatenate([ib, jnp.zeros((e_pad - e,), jnp.int32)])
    ia = ia.reshape(NW, nch, CH)
    ib = ib.reshape(NW, nch, CH)

    partials = _sc_xor_count(nch)(ptab, ia, ib)
    partials = (
        partials.reshape(NW, 32, LANES)
        .transpose(0, 2, 1)[:, :W, :]
        .reshape(NW, D)
    )

    scores = pl.pallas_call(
        _tc_reduce_am,
        out_shape=jax.ShapeDtypeStruct((1, am_weight.shape[0]), jnp.float32),
    )(partials, u, am_weight)
    return scores
